# baseline (speedup 1.0000x reference)
import jax
import jax.numpy as jnp
from jax.experimental import pallas as pl
from jax.experimental.pallas import tpu as pltpu
from jax.experimental.pallas import tpu_sc as plsc

B, S, D, K = 4, 4096, 4096, 8
BS = 512
NS = S // BS
NEG_INF = float("-inf")


def _scores_topk_kernel(x_ref, w_ref, b_ref, idx_ref, s_scratch):
    j = pl.program_id(1)
    xb = x_ref[0].astype(jnp.bfloat16).astype(jnp.float32)
    wb = w_ref[...].astype(jnp.bfloat16).astype(jnp.float32)
    v = xb * wb
    w_ = D
    while w_ > 128:
        h = w_ // 2
        v = v[:, :h] + v[:, h:w_]
        w_ = h
    part = jnp.sum(v, axis=1)
    s_scratch[0, pl.ds(j * BS, BS)] = part + b_ref[0, 0]

    @pl.when(j == NS - 1)
    def _():
        iota = jax.lax.broadcasted_iota(jnp.int32, (1, S), 1)
        kio = jax.lax.broadcasted_iota(jnp.int32, (1, K), 1)
        sv = s_scratch[0, :][None, :]
        idxs = jnp.zeros((1, K), jnp.int32)
        for k in range(K):
            m = jnp.max(sv)
            idx = jnp.min(jnp.where(sv == m, iota, S))
            idxs = jnp.where(kio == k, idx, idxs)
            sv = jnp.where(iota == idx, NEG_INF, sv)
        idx_ref[...] = idxs.reshape(1, 1, K)


def _sc_gather(idx_rows, x):
    n = B * K
    half = n // 2

    @pl.kernel(
        out_type=jax.ShapeDtypeStruct((B, K, D), jnp.float32),
        mesh=plsc.ScalarSubcoreMesh(axis_name="core", num_cores=2),
        scratch_types=[pltpu.SMEM((n,), jnp.int32),
                       pltpu.SemaphoreType.DMA],
    )
    def _k(idx_ref, x_ref, o_ref, idx_smem, sem):
        core = jax.lax.axis_index("core")
        pltpu.async_copy(idx_ref, idx_smem, sem).wait()

        def _copy(i):
            t = core * half + i
            b = t // K
            kk = jax.lax.rem(t, K)
            row = idx_smem[t]
            return pltpu.make_async_copy(
                x_ref.at[b, pl.ds(row, 1), :],
                o_ref.at[b, pl.ds(kk, 1), :],
                sem)

        @pl.loop(0, half)
        def _(i):
            _copy(i).start()

        @pl.loop(0, half)
        def _(i):
            _copy(i).wait()

    return _k(idx_rows, x)


def kernel(x, W, b):
    idx3 = pl.pallas_call(
        _scores_topk_kernel,
        grid=(B, NS),
        in_specs=[
            pl.BlockSpec((1, BS, D), lambda bi, j: (bi, j, 0)),
            pl.BlockSpec((1, D), lambda bi, j: (0, 0)),
            pl.BlockSpec((1, 1), lambda bi, j: (0, 0)),
        ],
        out_specs=pl.BlockSpec((1, 1, K), lambda bi, j: (bi, 0, 0)),
        out_shape=jax.ShapeDtypeStruct((B, 1, K), jnp.int32),
        scratch_shapes=[pltpu.VMEM((1, S), jnp.float32)],
        compiler_params=pltpu.CompilerParams(
            dimension_semantics=("parallel", "arbitrary"),
        ),
    )(x, W, b.reshape(1, 1))

    top_k_indices = idx3.reshape(B, K)
    idx_rows = top_k_indices.reshape(B * K)

    x_top_k = _sc_gather(idx_rows, x)

    return (x_top_k, top_k_indices[:, :, None])
